# baseline (speedup 1.0000x reference)
import jax
import jax.numpy as jnp
from jax.experimental import pallas as pl
from jax.experimental.pallas import tpu as pltpu

N_ROWS = 4096
N_COLS = 1000
BC = 512
GRID = N_ROWS // BC


def _cov_kernel(p_ref, t_ref, out_ref):
    p = p_ref[...]
    t = t_ref[...]
    masked = jnp.where(t > 0, p, jnp.inf)
    colmin = jnp.min(masked, axis=0, keepdims=True)
    cov = jnp.sum((p >= colmin).astype(jnp.float32), axis=0)
    cov = jnp.where(jnp.isfinite(colmin[0, :]), cov, 0.0)
    total = jnp.sum(cov)

    i = pl.program_id(0)

    @pl.when(i == 0)
    def _():
        out_ref[0] = 0.0

    out_ref[0] += total

    @pl.when(i == GRID - 1)
    def _():
        out_ref[0] = out_ref[0] * (1.0 / N_ROWS)


def kernel(predict_probs, true_labels):
    p = predict_probs.T
    t = true_labels.T
    out = pl.pallas_call(
        _cov_kernel,
        grid=(GRID,),
        in_specs=[
            pl.BlockSpec((N_COLS, BC), lambda i: (0, i)),
            pl.BlockSpec((N_COLS, BC), lambda i: (0, i)),
        ],
        out_specs=pl.BlockSpec(memory_space=pltpu.SMEM),
        out_shape=jax.ShapeDtypeStruct((1,), jnp.float32),
    )(p, t)
    return out[0]


# speedup vs baseline: 1.0965x; 1.0965x over previous
import jax
import jax.numpy as jnp
from jax.experimental import pallas as pl
from jax.experimental.pallas import tpu as pltpu

N_ROWS = 4096
N_COLS = 1000
BC = 1024
GRID = N_ROWS // BC


def _cov_kernel(p_ref, t_ref, out_ref):
    p = p_ref[...]
    t = t_ref[...]
    masked = jnp.where(t > 0, p, jnp.inf)
    colmin = jnp.min(masked, axis=0, keepdims=True)
    cov = jnp.sum((p >= colmin).astype(jnp.float32), axis=0)
    cov = jnp.where(jnp.isfinite(colmin[0, :]), cov, 0.0)
    total = jnp.sum(cov)

    i = pl.program_id(0)

    @pl.when(i == 0)
    def _():
        out_ref[0] = 0.0

    out_ref[0] += total

    @pl.when(i == GRID - 1)
    def _():
        out_ref[0] = out_ref[0] * (1.0 / N_ROWS)


def kernel(predict_probs, true_labels):
    p = predict_probs.T
    t = true_labels.T
    out = pl.pallas_call(
        _cov_kernel,
        grid=(GRID,),
        in_specs=[
            pl.BlockSpec((N_COLS, BC), lambda i: (0, i)),
            pl.BlockSpec((N_COLS, BC), lambda i: (0, i)),
        ],
        out_specs=pl.BlockSpec(memory_space=pltpu.SMEM),
        out_shape=jax.ShapeDtypeStruct((1,), jnp.float32),
    )(p, t)
    return out[0]
